# baseline (speedup 1.0000x reference)
import functools

import jax
import jax.numpy as jnp
from jax import lax
from jax.experimental import pallas as pl
from jax.experimental.pallas import tpu as pltpu
from jax.experimental.pallas import tpu_sc as plsc

N = 768
D = 128
E_Q = 49152

_SC_INFO = plsc.get_sparse_core_info()
_NC = _SC_INFO.num_cores
_NS = _SC_INFO.num_subcores
_NW = _NC * _NS
_EPW = E_Q // _NW
_CH = 128
_NCH = _EPW // _CH


def _dense_body(z_ref, ei_ref, w1_ref, b1_ref, w2_ref, b2_ref, bias_ref,
                g_ref, fidx_ref, h2_scr):
    k = pl.program_id(0)

    @pl.when(k == 0)
    def _prologue():
        z = z_ref[...]
        xw1 = jnp.dot(z, w1_ref[...], preferred_element_type=jnp.float32)
        m1 = jnp.sum(xw1, axis=0, keepdims=True) * (1.0 / N)
        h = jnp.maximum(z + m1 + b1_ref[...], 0.0)
        xw2 = jnp.dot(h, w2_ref[...], preferred_element_type=jnp.float32)
        m2 = jnp.sum(xw2, axis=0, keepdims=True) * (1.0 / N)
        h2_scr[...] = h + m2 + b2_ref[...]
        ei = ei_ref[...]
        fidx_ref[...] = ei[0, :] * N + ei[1, :]

    h2 = h2_scr[...]
    blk = h2_scr[pl.ds(k * (N // 6), N // 6), :]
    gram = -(lax.dot_general(
        blk, h2, (((1,), (1,)), ((), ())),
        preferred_element_type=jnp.float32) + bias_ref[0, 0])
    g_ref[...] = gram.reshape(N * N // (6 * D), D)


def _dense(z, edge_index, W1, b1, W2, b2, bias):
    return pl.pallas_call(
        _dense_body,
        grid=(6,),
        in_specs=[
            pl.BlockSpec((N, D), lambda k: (0, 0)),
            pl.BlockSpec((2, E_Q), lambda k: (0, 0)),
            pl.BlockSpec((D, D), lambda k: (0, 0)),
            pl.BlockSpec((1, D), lambda k: (0, 0)),
            pl.BlockSpec((D, D), lambda k: (0, 0)),
            pl.BlockSpec((1, D), lambda k: (0, 0)),
            pl.BlockSpec((1, 1), lambda k: (0, 0)),
        ],
        out_specs=(
            pl.BlockSpec((N * N // (6 * D), D), lambda k: (k, 0)),
            pl.BlockSpec((E_Q,), lambda k: (0,)),
        ),
        out_shape=(
            jax.ShapeDtypeStruct((N * N // D, D), jnp.float32),
            jax.ShapeDtypeStruct((E_Q,), jnp.int32),
        ),
        scratch_shapes=[pltpu.VMEM((N, D), jnp.float32)],
    )(z, edge_index, W1, b1.reshape(1, D), W2, b2.reshape(1, D),
      bias.reshape(1, 1).astype(jnp.float32))


def _score_body(g_hbm, fidx_hbm, out_hbm, fidx_v, gat_v, res_v, sem):
    wid = lax.axis_index("s") * _NC + lax.axis_index("c")
    base = wid * _EPW
    pltpu.sync_copy(fidx_hbm.at[pl.ds(base, _EPW)], fidx_v)
    copies = [
        pltpu.async_copy(g_hbm.at[fidx_v.at[pl.ds(c * _CH, _CH)]],
                         gat_v.at[pl.ds(c * _CH, _CH)], sem)
        for c in range(_NCH)
    ]
    outs = []
    for c in range(_NCH):
        copies[c].wait()
        for u in range(_CH // 16):
            o = c * _CH + u * 16
            res_v[pl.ds(o, 16)] = 1.0 / (1.0 + jnp.exp(gat_v[pl.ds(o, 16)]))
        outs.append(
            pltpu.async_copy(res_v.at[pl.ds(c * _CH, _CH)],
                             out_hbm.at[pl.ds(base + c * _CH, _CH)], sem))
    for o in outs:
        o.wait()


@functools.partial(jax.jit, static_argnames=())
def _score(g_flat, fidx):
    mesh = plsc.VectorSubcoreMesh(core_axis_name="c", subcore_axis_name="s")
    out = pl.kernel(
        _score_body,
        out_type=jax.ShapeDtypeStruct((E_Q,), jnp.float32),
        mesh=mesh,
        compiler_params=pltpu.CompilerParams(
            use_tc_tiling_on_sc=False, needs_layout_passes=False),
        scratch_types=[
            pltpu.VMEM((_EPW,), jnp.int32),
            pltpu.VMEM((_EPW,), jnp.float32),
            pltpu.VMEM((_EPW,), jnp.float32),
            pltpu.SemaphoreType.DMA,
        ],
    )(g_flat, fidx)
    return out


def kernel(z, edge_index, W1, b1, W2, b2, bias):
    g, fidx = _dense(z, edge_index, W1, b1, W2, b2, bias)
    return _score(g.reshape(N * N), fidx)


# speedup vs baseline: 1.0422x; 1.0422x over previous
import functools

import jax
import jax.numpy as jnp
from jax import lax
from jax.experimental import pallas as pl
from jax.experimental.pallas import tpu as pltpu
from jax.experimental.pallas import tpu_sc as plsc

N = 768
D = 128
E_Q = 49152

_SC_INFO = plsc.get_sparse_core_info()
_NC = _SC_INFO.num_cores
_NS = _SC_INFO.num_subcores
_NW = _NC * _NS
_EPW = E_Q // _NW
_CH = 128
_NCH = _EPW // _CH


def _dense_body(z_ref, ei_ref, w1_ref, b1_ref, w2_ref, b2_ref, bias_ref,
                g_ref, fidx_ref):
    z = z_ref[...]
    xw1 = jnp.dot(z, w1_ref[...], preferred_element_type=jnp.float32)
    m1 = jnp.sum(xw1, axis=0, keepdims=True) * (1.0 / N)
    h = jnp.maximum(z + m1 + b1_ref[...], 0.0)
    xw2 = jnp.dot(h, w2_ref[...], preferred_element_type=jnp.float32)
    m2 = jnp.sum(xw2, axis=0, keepdims=True) * (1.0 / N)
    h2 = h + m2 + b2_ref[...]
    gram = -(lax.dot_general(
        h2, h2, (((1,), (1,)), ((), ())),
        preferred_element_type=jnp.float32) + bias_ref[0, 0])
    g_ref[...] = gram.reshape(N * N // D, D)
    ei = ei_ref[...]
    fidx_ref[...] = ei[0, :] * N + ei[1, :]


def _dense(z, edge_index, W1, b1, W2, b2, bias):
    return pl.pallas_call(
        _dense_body,
        out_shape=(
            jax.ShapeDtypeStruct((N * N // D, D), jnp.float32),
            jax.ShapeDtypeStruct((E_Q,), jnp.int32),
        ),
    )(z, edge_index, W1, b1.reshape(1, D), W2, b2.reshape(1, D),
      bias.reshape(1, 1).astype(jnp.float32))


def _score_body(g_hbm, fidx_hbm, out_hbm, fidx_v, gat_v, res_v, sem):
    wid = lax.axis_index("s") * _NC + lax.axis_index("c")
    base = wid * _EPW
    pltpu.sync_copy(fidx_hbm.at[pl.ds(base, _EPW)], fidx_v)
    copies = [
        pltpu.async_copy(g_hbm.at[fidx_v.at[pl.ds(c * _CH, _CH)]],
                         gat_v.at[pl.ds(c * _CH, _CH)], sem)
        for c in range(_NCH)
    ]
    for c in range(_NCH):
        copies[c].wait()
        for u in range(_CH // 16):
            o = c * _CH + u * 16
            res_v[pl.ds(o, 16)] = 1.0 / (1.0 + jnp.exp(gat_v[pl.ds(o, 16)]))
    pltpu.sync_copy(res_v, out_hbm.at[pl.ds(base, _EPW)])


@functools.partial(jax.jit, static_argnames=())
def _score(g_flat, fidx):
    mesh = plsc.VectorSubcoreMesh(core_axis_name="c", subcore_axis_name="s")
    out = pl.kernel(
        _score_body,
        out_type=jax.ShapeDtypeStruct((E_Q,), jnp.float32),
        mesh=mesh,
        compiler_params=pltpu.CompilerParams(
            use_tc_tiling_on_sc=False, needs_layout_passes=False),
        scratch_types=[
            pltpu.VMEM((_EPW,), jnp.int32),
            pltpu.VMEM((_EPW,), jnp.float32),
            pltpu.VMEM((_EPW,), jnp.float32),
            pltpu.SemaphoreType.DMA,
        ],
    )(g_flat, fidx)
    return out


def kernel(z, edge_index, W1, b1, W2, b2, bias):
    g, fidx = _dense(z, edge_index, W1, b1, W2, b2, bias)
    return _score(g.reshape(N * N), fidx)
